# Initial kernel scaffold; baseline (speedup 1.0000x reference)
import jax
import jax.numpy as jnp
from jax.experimental import pallas as pl


def kernel(labels, bboxes, anchors):
    raise NotImplementedError("write your pallas kernel here")


# speedup vs baseline: 9.0279x; 9.0279x over previous
import functools

import jax
import jax.numpy as jnp
from jax import lax
from jax.experimental import pallas as pl
from jax.experimental.pallas import tpu as pltpu
from jax.experimental.pallas import tpu_sc as plsc

A = 32736
AP = 32768
B = 8
M = 100
MP = 112
NCORES = 2
NSUB = 16
PT = AP // NSUB
NCH = PT // 16
IMGS = B // NCORES
IOU_THR = 0.3


def _body(anc_hbm, box_hbm, lab_hbm,
          scores_hbm, obb_hbm,
          ax1_v, ay1_v, ax2_v, ay2_v, aar_v,
          bx1_v, by1_v, bx2_v, by2_v, bar_v,
          lab_v,
          cm_lane, cid_lane,
          cml_v, cil_v,
          shcm, shcid,
          shcm_v, shcid_v,
          cmg_v, cidg_v,
          rmax_v, ridx_v,
          score_v, obb_v):
    sub = lax.axis_index("s")
    core = lax.axis_index("c")
    base = sub * PT
    iota = lax.iota(jnp.int32, 16)

    pltpu.sync_copy(anc_hbm.at[0, pl.ds(base, PT)], ax1_v)
    pltpu.sync_copy(anc_hbm.at[1, pl.ds(base, PT)], ay1_v)
    pltpu.sync_copy(anc_hbm.at[2, pl.ds(base, PT)], ax2_v)
    pltpu.sync_copy(anc_hbm.at[3, pl.ds(base, PT)], ay2_v)

    def prep(ch, _):
        o = pl.ds(ch * 16, 16)
        cx = ax1_v[o]
        cy = ay1_v[o]
        w = ax2_v[o]
        h = ay2_v[o]
        x1 = cx - w * 0.5
        x2 = cx + w * 0.5
        y1 = cy - h * 0.5
        y2 = cy + h * 0.5
        ax1_v[o] = x1
        ax2_v[o] = x2
        ay1_v[o] = y1
        ay2_v[o] = y2
        aar_v[o] = (x2 - x1) * (y2 - y1)
        return 0

    lax.fori_loop(0, NCH, prep, 0)

    for b in range(IMGS):
        img = core * IMGS + b

        pltpu.sync_copy(box_hbm.at[img, 0], bx1_v)
        pltpu.sync_copy(box_hbm.at[img, 1], by1_v)
        pltpu.sync_copy(box_hbm.at[img, 2], bx2_v)
        pltpu.sync_copy(box_hbm.at[img, 3], by2_v)
        pltpu.sync_copy(lab_hbm.at[img], lab_v)
        for tc in range(MP // 16):
            o = pl.ds(tc * 16, 16)
            bar_v[o] = (bx2_v[o] - bx1_v[o]) * (by2_v[o] - by1_v[o])

        def zero(t, _):
            o = pl.ds(t * 16, 16)
            cm_lane[o] = jnp.zeros((16,), jnp.float32)
            cid_lane[o] = jnp.zeros((16,), jnp.int32)
            return 0

        lax.fori_loop(0, M, zero, 0)

        def chunk_body(ch, _):
            o = pl.ds(ch * 16, 16)
            x1 = ax1_v[o]
            y1 = ay1_v[o]
            x2 = ax2_v[o]
            y2 = ay2_v[o]
            aa = aar_v[o]
            ids = base + ch * 16 + iota

            def t_body(t, carry):
                rmax, ridx = carry
                ts = t + jnp.zeros((16,), jnp.int32)
                b1 = plsc.load_gather(bx1_v, [ts])
                b2 = plsc.load_gather(by1_v, [ts])
                b3 = plsc.load_gather(bx2_v, [ts])
                b4 = plsc.load_gather(by2_v, [ts])
                ab = plsc.load_gather(bar_v, [ts])
                wx = jnp.maximum(jnp.minimum(x2, b3) - jnp.maximum(x1, b1), 0.0)
                wy = jnp.maximum(jnp.minimum(y2, b4) - jnp.maximum(y1, b2), 0.0)
                inter = wx * wy
                iou = inter / (aa + ab - inter)
                m = iou > rmax
                rmax = jnp.where(m, iou, rmax)
                ridx = jnp.where(m, t, ridx)
                ot = pl.ds(t * 16, 16)
                cm = cm_lane[ot]
                ci = cid_lane[ot]
                m2 = iou > cm
                cm_lane[ot] = jnp.where(m2, iou, cm)
                cid_lane[ot] = jnp.where(m2, ids, ci)
                return (rmax, ridx)

            rmax, ridx = lax.fori_loop(
                0, M, t_body,
                (jnp.zeros((16,), jnp.float32), jnp.zeros((16,), jnp.int32)))
            rmax_v[o] = rmax
            ridx_v[o] = ridx
            return 0

        lax.fori_loop(0, NCH, chunk_body, 0)

        for tc in range(MP // 16):
            best = jnp.full((16,), -1.0, jnp.float32)
            besti = jnp.zeros((16,), jnp.int32)
            tvec = tc * 16 + iota
            valid = tvec < M
            for l in range(16):
                gidx = tvec * 16 + l
                gidx = jnp.where(valid, gidx, 0)
                v = plsc.load_gather(cm_lane, [gidx])
                iv = plsc.load_gather(cid_lane, [gidx])
                take = (v > best) | ((v == best) & (iv < besti))
                best = jnp.where(take, v, best)
                besti = jnp.where(take, iv, besti)
            o = pl.ds(tc * 16, 16)
            cml_v[o] = jnp.where(valid, best, 0.0)
            cil_v[o] = jnp.where(valid, besti, 0)

        pltpu.sync_copy(cml_v, shcm.at[pl.ds(sub * MP, MP)])
        pltpu.sync_copy(cil_v, shcid.at[pl.ds(sub * MP, MP)])
        plsc.subcore_barrier()
        pltpu.sync_copy(shcm, shcm_v)
        pltpu.sync_copy(shcid, shcid_v)
        plsc.subcore_barrier()

        for tc in range(MP // 16):
            g = jnp.full((16,), -1.0, jnp.float32)
            gi = jnp.zeros((16,), jnp.int32)
            for r in range(NSUB):
                o = pl.ds(r * MP + tc * 16, 16)
                v = shcm_v[o]
                iv = shcid_v[o]
                take = (v > g) | ((v == g) & (iv < gi))
                g = jnp.where(take, v, g)
                gi = jnp.where(take, iv, gi)
            o = pl.ds(tc * 16, 16)
            cmg_v[o] = g
            cidg_v[o] = gi

        for tc in range(MP // 16):
            o = pl.ds(tc * 16, 16)
            av = cidg_v[o]
            cmv = cmg_v[o]
            tv = tc * 16 + iota
            local = av - base
            inr = (local >= 0) & (local < PT) & (tv < M)
            locc = jnp.where(inr, local, 0)
            for l in range(16):
                msk = inr & (iota == l)
                plsc.store_scatter(rmax_v, [locc], cmv, mask=msk)
                plsc.store_scatter(ridx_v, [locc], tv, mask=msk)

        def out_body(ch, _):
            o = pl.ds(ch * 16, 16)
            rmax = rmax_v[o]
            bidx = ridx_v[o]
            cmat = plsc.load_gather(cmg_v, [bidx])
            labg = plsc.load_gather(lab_v, [bidx])
            gx1 = plsc.load_gather(bx1_v, [bidx])
            gy1 = plsc.load_gather(by1_v, [bidx])
            gx2 = plsc.load_gather(bx2_v, [bidx])
            gy2 = plsc.load_gather(by2_v, [bidx])
            denom = jnp.where(cmat < IOU_THR, IOU_THR, cmat)
            mia = jnp.where(rmax < IOU_THR / 2.0, 0.0, rmax)
            sc = mia / denom
            sc = jnp.where(labg <= 0, 0.0, sc)
            score_v[o] = sc
            pos = (ch * 16 + iota) * 4
            plsc.store_scatter(obb_v, [pos], gx1)
            plsc.store_scatter(obb_v, [pos + 1], gy1)
            plsc.store_scatter(obb_v, [pos + 2], gx2)
            plsc.store_scatter(obb_v, [pos + 3], gy2)
            return 0

        lax.fori_loop(0, NCH, out_body, 0)

        pltpu.sync_copy(score_v, scores_hbm.at[img, pl.ds(base, PT)])
        pltpu.sync_copy(obb_v, obb_hbm.at[img, pl.ds(base * 4, PT * 4)])


@jax.jit
def _run(anchorsT, boxesT, labelsP):
    mesh = plsc.VectorSubcoreMesh(core_axis_name="c", subcore_axis_name="s",
                                  num_cores=NCORES, num_subcores=NSUB)
    f32 = jnp.float32
    i32 = jnp.int32
    kern = pl.kernel(
        _body,
        out_type=(jax.ShapeDtypeStruct((B, AP), f32),
                  jax.ShapeDtypeStruct((B, AP * 4), f32)),
        mesh=mesh,
        compiler_params=pltpu.CompilerParams(needs_layout_passes=False),
        scratch_types=(
            pltpu.VMEM((PT,), f32), pltpu.VMEM((PT,), f32),
            pltpu.VMEM((PT,), f32), pltpu.VMEM((PT,), f32),
            pltpu.VMEM((PT,), f32),
            pltpu.VMEM((MP,), f32), pltpu.VMEM((MP,), f32),
            pltpu.VMEM((MP,), f32), pltpu.VMEM((MP,), f32),
            pltpu.VMEM((MP,), f32),
            pltpu.VMEM((MP,), i32),
            pltpu.VMEM((M * 16,), f32), pltpu.VMEM((M * 16,), i32),
            pltpu.VMEM((MP,), f32), pltpu.VMEM((MP,), i32),
            pltpu.VMEM_SHARED((NSUB * MP,), f32),
            pltpu.VMEM_SHARED((NSUB * MP,), i32),
            pltpu.VMEM((NSUB * MP,), f32), pltpu.VMEM((NSUB * MP,), i32),
            pltpu.VMEM((MP,), f32), pltpu.VMEM((MP,), i32),
            pltpu.VMEM((PT,), f32), pltpu.VMEM((PT,), i32),
            pltpu.VMEM((PT,), f32), pltpu.VMEM((PT * 4,), f32),
        ),
    )
    return kern(anchorsT, boxesT, labelsP)


def kernel(labels, bboxes, anchors):
    anchorsT = jnp.pad(jnp.transpose(anchors, (1, 0)), ((0, 0), (0, AP - A)))
    boxesT = jnp.pad(jnp.transpose(bboxes, (0, 2, 1)),
                     ((0, 0), (0, 0), (0, MP - M)))
    labelsP = jnp.pad(labels.astype(jnp.int32), ((0, 0), (0, MP - M)))
    scores_p, obb_p = _run(anchorsT, boxesT, labelsP)
    obb = obb_p.reshape(B, AP, 4)
    return scores_p[:, :A], obb[:, :A, :]


# speedup vs baseline: 25.4260x; 2.8164x over previous
import functools

import jax
import jax.numpy as jnp
from jax import lax
from jax.experimental import pallas as pl
from jax.experimental.pallas import tpu as pltpu
from jax.experimental.pallas import tpu_sc as plsc

A = 32736
AP = 32768
B = 8
M = 100
MP = 112
NCORES = 2
NSUB = 16
PT = AP // NSUB
NCH = PT // 16
IMGS = B // NCORES
IOU_THR = 0.3


def _body(anc_hbm, box_hbm, lab_hbm,
          scores_hbm, obb_hbm,
          ax1_v, ay1_v, ax2_v, ay2_v, aar_v,
          bx1_v, by1_v, bx2_v, by2_v, bar_v,
          lab_v,
          cm_lane, cid_lane,
          cml_v, cil_v,
          shcm, shcid,
          shcm_v, shcid_v,
          cmg_v, cidg_v,
          rmax_v, ridx_v,
          score_v, obb_v):
    sub = lax.axis_index("s")
    core = lax.axis_index("c")
    base = sub * PT
    iota = lax.iota(jnp.int32, 16)

    pltpu.sync_copy(anc_hbm.at[0, pl.ds(base, PT)], ax1_v)
    pltpu.sync_copy(anc_hbm.at[1, pl.ds(base, PT)], ay1_v)
    pltpu.sync_copy(anc_hbm.at[2, pl.ds(base, PT)], ax2_v)
    pltpu.sync_copy(anc_hbm.at[3, pl.ds(base, PT)], ay2_v)

    def prep(ch, _):
        o = pl.ds(ch * 16, 16)
        cx = ax1_v[o]
        cy = ay1_v[o]
        w = ax2_v[o]
        h = ay2_v[o]
        x1 = cx - w * 0.5
        x2 = cx + w * 0.5
        y1 = cy - h * 0.5
        y2 = cy + h * 0.5
        ax1_v[o] = x1
        ax2_v[o] = x2
        ay1_v[o] = y1
        ay2_v[o] = y2
        aar_v[o] = (x2 - x1) * (y2 - y1)
        return 0

    lax.fori_loop(0, NCH, prep, 0)

    for b in range(IMGS):
        img = core * IMGS + b

        pltpu.sync_copy(box_hbm.at[img, 0], bx1_v)
        pltpu.sync_copy(box_hbm.at[img, 1], by1_v)
        pltpu.sync_copy(box_hbm.at[img, 2], bx2_v)
        pltpu.sync_copy(box_hbm.at[img, 3], by2_v)
        pltpu.sync_copy(lab_hbm.at[img], lab_v)
        for tc in range(MP // 16):
            o = pl.ds(tc * 16, 16)
            bar_v[o] = (bx2_v[o] - bx1_v[o]) * (by2_v[o] - by1_v[o])

        def zero(t, _):
            o = pl.ds(t * 16, 16)
            cm_lane[o] = jnp.zeros((16,), jnp.float32)
            cid_lane[o] = jnp.zeros((16,), jnp.int32)
            return 0

        lax.fori_loop(0, M, zero, 0)

        def chunk_body(ch, _):
            o = pl.ds(ch * 16, 16)
            x1 = ax1_v[o]
            y1 = ay1_v[o]
            x2 = ax2_v[o]
            y2 = ay2_v[o]
            aa = aar_v[o]
            ids = base + ch * 16 + iota

            def t_body(t, carry):
                rmax, ridx = carry
                ts = t + jnp.zeros((16,), jnp.int32)
                b1 = plsc.load_gather(bx1_v, [ts])
                b2 = plsc.load_gather(by1_v, [ts])
                b3 = plsc.load_gather(bx2_v, [ts])
                b4 = plsc.load_gather(by2_v, [ts])
                ab = plsc.load_gather(bar_v, [ts])
                wx = jnp.maximum(jnp.minimum(x2, b3) - jnp.maximum(x1, b1), 0.0)
                wy = jnp.maximum(jnp.minimum(y2, b4) - jnp.maximum(y1, b2), 0.0)
                inter = wx * wy
                iou = inter / (aa + ab - inter)
                m = iou > rmax
                rmax = jnp.where(m, iou, rmax)
                ridx = jnp.where(m, t, ridx)
                ot = pl.ds(t * 16, 16)
                cm = cm_lane[ot]
                ci = cid_lane[ot]
                m2 = iou > cm
                cm_lane[ot] = jnp.where(m2, iou, cm)
                cid_lane[ot] = jnp.where(m2, ids, ci)
                return (rmax, ridx)

            rmax, ridx = plsc.parallel_loop(
                0, M, carry=(jnp.zeros((16,), jnp.float32),
                             jnp.zeros((16,), jnp.int32)),
                unroll=4)(t_body)
            rmax_v[o] = rmax
            ridx_v[o] = ridx
            return 0

        lax.fori_loop(0, NCH, chunk_body, 0)

        for tc in range(MP // 16):
            best = jnp.full((16,), -1.0, jnp.float32)
            besti = jnp.zeros((16,), jnp.int32)
            tvec = tc * 16 + iota
            valid = tvec < M
            for l in range(16):
                gidx = tvec * 16 + l
                gidx = jnp.where(valid, gidx, 0)
                v = plsc.load_gather(cm_lane, [gidx])
                iv = plsc.load_gather(cid_lane, [gidx])
                take = (v > best) | ((v == best) & (iv < besti))
                best = jnp.where(take, v, best)
                besti = jnp.where(take, iv, besti)
            o = pl.ds(tc * 16, 16)
            cml_v[o] = jnp.where(valid, best, 0.0)
            cil_v[o] = jnp.where(valid, besti, 0)

        pltpu.sync_copy(cml_v, shcm.at[pl.ds(sub * MP, MP)])
        pltpu.sync_copy(cil_v, shcid.at[pl.ds(sub * MP, MP)])
        plsc.subcore_barrier()
        pltpu.sync_copy(shcm, shcm_v)
        pltpu.sync_copy(shcid, shcid_v)
        plsc.subcore_barrier()

        for tc in range(MP // 16):
            g = jnp.full((16,), -1.0, jnp.float32)
            gi = jnp.zeros((16,), jnp.int32)
            for r in range(NSUB):
                o = pl.ds(r * MP + tc * 16, 16)
                v = shcm_v[o]
                iv = shcid_v[o]
                take = (v > g) | ((v == g) & (iv < gi))
                g = jnp.where(take, v, g)
                gi = jnp.where(take, iv, gi)
            o = pl.ds(tc * 16, 16)
            cmg_v[o] = g
            cidg_v[o] = gi

        for tc in range(MP // 16):
            o = pl.ds(tc * 16, 16)
            av = cidg_v[o]
            cmv = cmg_v[o]
            tv = tc * 16 + iota
            local = av - base
            inr = (local >= 0) & (local < PT) & (tv < M)
            locc = jnp.where(inr, local, 0)
            for l in range(16):
                msk = inr & (iota == l)
                plsc.store_scatter(rmax_v, [locc], cmv, mask=msk)
                plsc.store_scatter(ridx_v, [locc], tv, mask=msk)

        def out_body(ch, _):
            o = pl.ds(ch * 16, 16)
            rmax = rmax_v[o]
            bidx = ridx_v[o]
            cmat = plsc.load_gather(cmg_v, [bidx])
            labg = plsc.load_gather(lab_v, [bidx])
            gx1 = plsc.load_gather(bx1_v, [bidx])
            gy1 = plsc.load_gather(by1_v, [bidx])
            gx2 = plsc.load_gather(bx2_v, [bidx])
            gy2 = plsc.load_gather(by2_v, [bidx])
            denom = jnp.where(cmat < IOU_THR, IOU_THR, cmat)
            mia = jnp.where(rmax < IOU_THR / 2.0, 0.0, rmax)
            sc = mia / denom
            sc = jnp.where(labg <= 0, 0.0, sc)
            score_v[o] = sc
            pos = (ch * 16 + iota) * 4
            plsc.store_scatter(obb_v, [pos], gx1)
            plsc.store_scatter(obb_v, [pos + 1], gy1)
            plsc.store_scatter(obb_v, [pos + 2], gx2)
            plsc.store_scatter(obb_v, [pos + 3], gy2)
            return 0

        lax.fori_loop(0, NCH, out_body, 0)

        pltpu.sync_copy(score_v, scores_hbm.at[img, pl.ds(base, PT)])
        pltpu.sync_copy(obb_v, obb_hbm.at[img, pl.ds(base * 4, PT * 4)])


@jax.jit
def _run(anchorsT, boxesT, labelsP):
    mesh = plsc.VectorSubcoreMesh(core_axis_name="c", subcore_axis_name="s",
                                  num_cores=NCORES, num_subcores=NSUB)
    f32 = jnp.float32
    i32 = jnp.int32
    kern = pl.kernel(
        _body,
        out_type=(jax.ShapeDtypeStruct((B, AP), f32),
                  jax.ShapeDtypeStruct((B, AP * 4), f32)),
        mesh=mesh,
        compiler_params=pltpu.CompilerParams(needs_layout_passes=False),
        scratch_types=(
            pltpu.VMEM((PT,), f32), pltpu.VMEM((PT,), f32),
            pltpu.VMEM((PT,), f32), pltpu.VMEM((PT,), f32),
            pltpu.VMEM((PT,), f32),
            pltpu.VMEM((MP,), f32), pltpu.VMEM((MP,), f32),
            pltpu.VMEM((MP,), f32), pltpu.VMEM((MP,), f32),
            pltpu.VMEM((MP,), f32),
            pltpu.VMEM((MP,), i32),
            pltpu.VMEM((M * 16,), f32), pltpu.VMEM((M * 16,), i32),
            pltpu.VMEM((MP,), f32), pltpu.VMEM((MP,), i32),
            pltpu.VMEM_SHARED((NSUB * MP,), f32),
            pltpu.VMEM_SHARED((NSUB * MP,), i32),
            pltpu.VMEM((NSUB * MP,), f32), pltpu.VMEM((NSUB * MP,), i32),
            pltpu.VMEM((MP,), f32), pltpu.VMEM((MP,), i32),
            pltpu.VMEM((PT,), f32), pltpu.VMEM((PT,), i32),
            pltpu.VMEM((PT,), f32), pltpu.VMEM((PT * 4,), f32),
        ),
    )
    return kern(anchorsT, boxesT, labelsP)


def kernel(labels, bboxes, anchors):
    anchorsT = jnp.pad(jnp.transpose(anchors, (1, 0)), ((0, 0), (0, AP - A)))
    boxesT = jnp.pad(jnp.transpose(bboxes, (0, 2, 1)),
                     ((0, 0), (0, 0), (0, MP - M)))
    labelsP = jnp.pad(labels.astype(jnp.int32), ((0, 0), (0, MP - M)))
    scores_p, obb_p = _run(anchorsT, boxesT, labelsP)
    obb = obb_p.reshape(B, AP, 4)
    return scores_p[:, :A], obb[:, :A, :]
